# baseline (speedup 1.0000x reference)
import jax
import jax.numpy as jnp
from jax import lax
from jax.experimental import pallas as pl
from jax.experimental.pallas import tpu as pltpu
from jax.experimental.pallas import tpu_sc as plsc

_B, _N, _C, _H, _W = 4, 4096, 256, 176, 176
_HW = _H * _W
_NWORKERS = 32
_PPW = _B * _N // _NWORKERS
_PC = 32
_NCH = _PPW // _PC
_ROWS = 4 * _PC
_LANES = 16
_SCALE = 2.5
_YOFF = 40.0


def _sc_body(bev_hbm, kp_hbm, out_hbm,
             rv0, rv1, ob0, ob1, idx_all, wbuf, kx_v, ky_v,
             semG0, semG1, semO0, semO1):
    wid = lax.axis_index("s") * 2 + lax.axis_index("c")
    b = wid // (_NWORKERS // _B)
    n0 = wid * _PPW
    nb0 = (wid % (_NWORKERS // _B)) * _PPW
    cell0 = b * _HW

    pltpu.sync_copy(kp_hbm.at[0, b, pl.ds(nb0, _PPW)], kx_v)
    pltpu.sync_copy(kp_hbm.at[1, b, pl.ds(nb0, _PPW)], ky_v)

    iota4 = lax.iota(jnp.int32, _LANES) * 4

    @plsc.parallel_loop(0, _PPW // _LANES, unroll=2)
    def _pre(i):
        s = pl.ds(i * _LANES, _LANES)
        x = kx_v[s] * _SCALE
        y = (ky_v[s] + _YOFF) * _SCALE
        x0 = jnp.maximum(x.astype(jnp.int32), 0)
        y0 = jnp.maximum(y.astype(jnp.int32), 0)
        x0c = jnp.minimum(x0, _W - 1)
        x1c = jnp.minimum(x0 + 1, _W - 1)
        y0c = jnp.minimum(y0, _H - 1)
        y1c = jnp.minimum(y0 + 1, _H - 1)
        xf0 = x0c.astype(jnp.float32)
        xf1 = x1c.astype(jnp.float32)
        yf0 = y0c.astype(jnp.float32)
        yf1 = y1c.astype(jnp.float32)
        widx = iota4 + i * (_LANES * 4)
        plsc.store_scatter(wbuf, [widx], (xf1 - x) * (yf1 - y))
        plsc.store_scatter(wbuf, [widx + 1], (xf1 - x) * (y - yf0))
        plsc.store_scatter(wbuf, [widx + 2], (x - xf0) * (yf1 - y))
        plsc.store_scatter(wbuf, [widx + 3], (x - xf0) * (y - yf0))
        j = i // 2
        col = (i % 2) * _LANES
        r0 = cell0 + y0c * _W
        r1 = cell0 + y1c * _W
        idx_all[j, pl.ds(0 * _PC + col, _LANES)] = r0 + x0c
        idx_all[j, pl.ds(1 * _PC + col, _LANES)] = r1 + x0c
        idx_all[j, pl.ds(2 * _PC + col, _LANES)] = r0 + x1c
        idx_all[j, pl.ds(3 * _PC + col, _LANES)] = r1 + x1c

    def _compute(rv, ob, a):
        @plsc.parallel_loop(0, _PC, unroll=1)
        def _pt(p):
            pg = a * _PC + p
            w4 = wbuf[pl.ds(pg * 4, _LANES)]
            w0 = w4[0]
            w1 = w4[1]
            w2 = w4[2]
            w3 = w4[3]
            for ch in range(_C // _LANES):
                s = pl.ds(ch * _LANES, _LANES)
                acc = rv[0 * _PC + p, s] * w0
                acc = acc + rv[1 * _PC + p, s] * w1
                acc = acc + rv[2 * _PC + p, s] * w2
                acc = acc + rv[3 * _PC + p, s] * w3
                ob[p, s] = acc

    pltpu.async_copy(bev_hbm.at[idx_all.at[0]], rv0, semG0)
    pltpu.async_copy(bev_hbm.at[idx_all.at[1]], rv1, semG1)

    def pair(m, carry):
        a = 2 * m

        @pl.when(m >= 1)
        def _():
            pltpu.make_async_copy(ob0, out_hbm.at[pl.ds(n0 + (a - 2) * _PC, _PC)], semO0).wait()
        pltpu.make_async_copy(bev_hbm.at[idx_all.at[a]], rv0, semG0).wait()
        _compute(rv0, ob0, a)
        pltpu.async_copy(ob0, out_hbm.at[pl.ds(n0 + a * _PC, _PC)], semO0)

        @pl.when(m < _NCH // 2 - 1)
        def _():
            pltpu.async_copy(bev_hbm.at[idx_all.at[a + 2]], rv0, semG0)

        @pl.when(m >= 1)
        def _():
            pltpu.make_async_copy(ob1, out_hbm.at[pl.ds(n0 + (a - 1) * _PC, _PC)], semO1).wait()
        pltpu.make_async_copy(bev_hbm.at[idx_all.at[a + 1]], rv1, semG1).wait()
        _compute(rv1, ob1, a + 1)
        pltpu.async_copy(ob1, out_hbm.at[pl.ds(n0 + (a + 1) * _PC, _PC)], semO1)

        @pl.when(m < _NCH // 2 - 1)
        def _():
            pltpu.async_copy(bev_hbm.at[idx_all.at[a + 3]], rv1, semG1)

        return carry

    lax.fori_loop(0, _NCH // 2, pair, 0)

    pltpu.make_async_copy(ob0, out_hbm.at[pl.ds(n0 + (_NCH - 2) * _PC, _PC)], semO0).wait()
    pltpu.make_async_copy(ob1, out_hbm.at[pl.ds(n0 + (_NCH - 1) * _PC, _PC)], semO1).wait()


def kernel(keypoints, bev_features, bev_stride):
    bev_rows = jnp.transpose(bev_features, (0, 2, 3, 1)).reshape(_B * _HW, _C)
    kp3 = jnp.transpose(keypoints, (2, 0, 1))
    mesh = plsc.VectorSubcoreMesh(core_axis_name="c", subcore_axis_name="s")
    k = pl.kernel(
        _sc_body,
        mesh=mesh,
        out_type=jax.ShapeDtypeStruct((_B * _N, _C), jnp.float32),
        compiler_params=pltpu.CompilerParams(
            needs_layout_passes=False,
            disable_bounds_checks=True,
            skip_device_barrier=True,
        ),
        scratch_types=[
            pltpu.VMEM((_ROWS, _C), jnp.float32),
            pltpu.VMEM((_ROWS, _C), jnp.float32),
            pltpu.VMEM((_PC, _C), jnp.float32),
            pltpu.VMEM((_PC, _C), jnp.float32),
            pltpu.VMEM((_NCH, _ROWS), jnp.int32),
            pltpu.VMEM((4 * _PPW + _LANES,), jnp.float32),
            pltpu.VMEM((_PPW,), jnp.float32),
            pltpu.VMEM((_PPW,), jnp.float32),
            pltpu.SemaphoreType.DMA,
            pltpu.SemaphoreType.DMA,
            pltpu.SemaphoreType.DMA,
            pltpu.SemaphoreType.DMA,
        ],
    )
    out2d = k(bev_rows, kp3)
    return out2d.reshape(_B, _N, _C)


# speedup vs baseline: 1.0018x; 1.0018x over previous
import jax
import jax.numpy as jnp
from jax import lax
from jax.experimental import pallas as pl
from jax.experimental.pallas import tpu as pltpu
from jax.experimental.pallas import tpu_sc as plsc

_B, _N, _C, _H, _W = 4, 4096, 256, 176, 176
_HW = _H * _W
_NWORKERS = 32
_PPW = _B * _N // _NWORKERS
_PC = 32
_NCH = _PPW // _PC
_ROWS = 4 * _PC
_LANES = 16
_SCALE = 2.5
_YOFF = 40.0


def _sc_body(bev_hbm, kp_hbm, out_hbm,
             rv0, rv1, ob0, ob1, idx_all, wbuf, kx_v, ky_v,
             semG0, semG1, semO0, semO1):
    wid = lax.axis_index("s") * 2 + lax.axis_index("c")
    b = wid // (_NWORKERS // _B)
    n0 = wid * _PPW
    nb0 = (wid % (_NWORKERS // _B)) * _PPW
    cell0 = b * _HW

    pltpu.sync_copy(kp_hbm.at[0, b, pl.ds(nb0, _PPW)], kx_v)
    pltpu.sync_copy(kp_hbm.at[1, b, pl.ds(nb0, _PPW)], ky_v)

    iota4 = lax.iota(jnp.int32, _LANES) * 4

    @plsc.parallel_loop(0, _PPW // _LANES, unroll=2)
    def _pre(i):
        s = pl.ds(i * _LANES, _LANES)
        x = kx_v[s] * _SCALE
        y = (ky_v[s] + _YOFF) * _SCALE
        x0 = jnp.maximum(x.astype(jnp.int32), 0)
        y0 = jnp.maximum(y.astype(jnp.int32), 0)
        x0c = jnp.minimum(x0, _W - 1)
        x1c = jnp.minimum(x0 + 1, _W - 1)
        y0c = jnp.minimum(y0, _H - 1)
        y1c = jnp.minimum(y0 + 1, _H - 1)
        xf0 = x0c.astype(jnp.float32)
        xf1 = x1c.astype(jnp.float32)
        yf0 = y0c.astype(jnp.float32)
        yf1 = y1c.astype(jnp.float32)
        widx = iota4 + i * (_LANES * 4)
        plsc.store_scatter(wbuf, [widx], (xf1 - x) * (yf1 - y))
        plsc.store_scatter(wbuf, [widx + 1], (xf1 - x) * (y - yf0))
        plsc.store_scatter(wbuf, [widx + 2], (x - xf0) * (yf1 - y))
        plsc.store_scatter(wbuf, [widx + 3], (x - xf0) * (y - yf0))
        j = i // 2
        col = (i % 2) * _LANES
        r0 = cell0 + y0c * _W
        r1 = cell0 + y1c * _W
        idx_all[j, pl.ds(0 * _PC + col, _LANES)] = r0 + x0c
        idx_all[j, pl.ds(1 * _PC + col, _LANES)] = r1 + x0c
        idx_all[j, pl.ds(2 * _PC + col, _LANES)] = r0 + x1c
        idx_all[j, pl.ds(3 * _PC + col, _LANES)] = r1 + x1c

    def _compute(rv, ob, a):
        @plsc.parallel_loop(0, _PC, unroll=1)
        def _pt(p):
            pg = a * _PC + p
            w4 = wbuf[pl.ds(pg * 4, _LANES)]
            w0 = w4[0]
            w1 = w4[1]
            w2 = w4[2]
            w3 = w4[3]
            for ch in range(_C // _LANES):
                s = pl.ds(ch * _LANES, _LANES)
                acc = rv[0 * _PC + p, s] * w0
                acc = acc + rv[1 * _PC + p, s] * w1
                acc = acc + rv[2 * _PC + p, s] * w2
                acc = acc + rv[3 * _PC + p, s] * w3
                ob[p, s] = acc

    pltpu.async_copy(bev_hbm.at[idx_all.at[0]], rv0, semG0)
    pltpu.async_copy(bev_hbm.at[idx_all.at[1]], rv1, semG1)

    def pair(m, carry):
        a = 2 * m

        @pl.when(m >= 1)
        def _():
            pltpu.make_async_copy(ob0, out_hbm.at[pl.ds(n0 + (a - 2) * _PC, _PC)], semO0).wait()
        pltpu.make_async_copy(bev_hbm.at[idx_all.at[a]], rv0, semG0).wait()
        _compute(rv0, ob0, a)
        pltpu.async_copy(ob0, out_hbm.at[pl.ds(n0 + a * _PC, _PC)], semO0)

        @pl.when(m < _NCH // 2 - 1)
        def _():
            pltpu.async_copy(bev_hbm.at[idx_all.at[a + 2]], rv0, semG0)

        @pl.when(m >= 1)
        def _():
            pltpu.make_async_copy(ob1, out_hbm.at[pl.ds(n0 + (a - 1) * _PC, _PC)], semO1).wait()
        pltpu.make_async_copy(bev_hbm.at[idx_all.at[a + 1]], rv1, semG1).wait()
        _compute(rv1, ob1, a + 1)
        pltpu.async_copy(ob1, out_hbm.at[pl.ds(n0 + (a + 1) * _PC, _PC)], semO1)

        @pl.when(m < _NCH // 2 - 1)
        def _():
            pltpu.async_copy(bev_hbm.at[idx_all.at[a + 3]], rv1, semG1)

        return carry

    lax.fori_loop(0, _NCH // 2, pair, 0)

    pltpu.make_async_copy(ob0, out_hbm.at[pl.ds(n0 + (_NCH - 2) * _PC, _PC)], semO0).wait()
    pltpu.make_async_copy(ob1, out_hbm.at[pl.ds(n0 + (_NCH - 1) * _PC, _PC)], semO1).wait()


def kernel(keypoints, bev_features, bev_stride):
    bev_rows = jnp.transpose(bev_features, (0, 2, 3, 1)).reshape(_B * _HW, _C)
    kp3 = jnp.transpose(keypoints, (2, 0, 1))
    mesh = plsc.VectorSubcoreMesh(core_axis_name="c", subcore_axis_name="s")
    k = pl.kernel(
        _sc_body,
        mesh=mesh,
        out_type=jax.ShapeDtypeStruct((_B * _N, _C), jnp.float32),
        compiler_params=pltpu.CompilerParams(needs_layout_passes=False),
        scratch_types=[
            pltpu.VMEM((_ROWS, _C), jnp.float32),
            pltpu.VMEM((_ROWS, _C), jnp.float32),
            pltpu.VMEM((_PC, _C), jnp.float32),
            pltpu.VMEM((_PC, _C), jnp.float32),
            pltpu.VMEM((_NCH, _ROWS), jnp.int32),
            pltpu.VMEM((4 * _PPW + _LANES,), jnp.float32),
            pltpu.VMEM((_PPW,), jnp.float32),
            pltpu.VMEM((_PPW,), jnp.float32),
            pltpu.SemaphoreType.DMA,
            pltpu.SemaphoreType.DMA,
            pltpu.SemaphoreType.DMA,
            pltpu.SemaphoreType.DMA,
        ],
    )
    out2d = k(bev_rows, kp3)
    return out2d.reshape(_B, _N, _C)
